# baseline (speedup 1.0000x reference)
import functools

import jax
import jax.numpy as jnp
from jax import lax
from jax.experimental import pallas as pl
from jax.experimental.pallas import tpu as pltpu
from jax.experimental.pallas import tpu_sc as plsc

_LANES = 16
_NC = 2
_NS = 16
_NW = _NC * _NS
_IDXW = 128
_BLK = 4096


def _proj_body(n, x_ref, w_ref, bp_ref, out_ref):
    x = x_ref[...]
    p = jnp.dot(x, w_ref[...], preferred_element_type=jnp.float32)
    out_ref[0:n, :] = p[:, 0:_LANES] + bp_ref[...]
    out_ref[n : 2 * n, :] = p[:, _LANES : 2 * _LANES]


def _make_tables(x, W, b_lin):
    n, d = x.shape
    a = W.shape[1]
    ws = jnp.pad(W[:d], ((0, 0), (0, _LANES - a)))
    wd = jnp.pad(W[d:], ((0, 0), (0, _LANES - a)))
    wfull = jnp.concatenate([ws, wd], axis=1)
    bp = jnp.full((1, _LANES), -1e30, dtype=jnp.float32)
    bp = bp.at[0, :a].set(b_lin.astype(jnp.float32))
    return pl.pallas_call(
        functools.partial(_proj_body, n),
        out_shape=jax.ShapeDtypeStruct((2 * n, _LANES), jnp.float32),
    )(x.astype(jnp.float32), wfull, bp)


def _gather_add_body(edges_per_w, chunk_rows, table_ref, si_ref, di_ref,
                     out_ref, si_v, di_v, a0_v, b0_v, a1_v, b1_v,
                     g0_sem, g1_sem, s0_sem, s1_sem):
    wid = lax.axis_index("s") * _NC + lax.axis_index("c")
    e0 = wid * edges_per_w
    pltpu.sync_copy(si_ref.at[pl.ds(e0, edges_per_w)], si_v)
    pltpu.sync_copy(di_ref.at[pl.ds(e0, edges_per_w)], di_v)
    cedges = chunk_rows * _IDXW
    nchunks = edges_per_w // cedges
    a_bufs, b_bufs = (a0_v, a1_v), (b0_v, b1_v)
    g_sems, s_sems = (g0_sem, g1_sem), (s0_sem, s1_sem)

    def fire(ch, slot):
        ds = []
        for j in range(chunk_rows):
            o = ch * cedges + j * _IDXW
            ds.append(pltpu.async_copy(
                table_ref.at[si_v.at[pl.ds(o, _IDXW)]],
                a_bufs[slot].at[pl.ds(j * _IDXW, _IDXW)], g_sems[slot]))
            ds.append(pltpu.async_copy(
                table_ref.at[di_v.at[pl.ds(o, _IDXW)]],
                b_bufs[slot].at[pl.ds(j * _IDXW, _IDXW)], g_sems[slot]))
        return ds

    pend_g = {0: fire(0, 0)}
    pend_s = {}
    for ch in range(nchunks):
        slot = ch % 2
        nxt = 1 - slot
        if ch + 1 < nchunks:
            if nxt in pend_s:
                pend_s.pop(nxt).wait()
            pend_g[nxt] = fire(ch + 1, nxt)
        for dsc in pend_g.pop(slot):
            dsc.wait()
        a_v, b_v = a_bufs[slot], b_bufs[slot]

        def _add(i, _):
            a_v[i, :] = a_v[i, :] + b_v[i, :]
            return 0

        lax.fori_loop(0, cedges, _add, 0, unroll=8)
        pend_s[slot] = pltpu.async_copy(
            a_v, out_ref.at[pl.ds(e0 + ch * cedges, cedges)], s_sems[slot])
    for dsc in pend_s.values():
        dsc.wait()


def _gather_add(table, si, di, ep, chunk_rows):
    edges_per_w = ep // _NW
    mesh = plsc.VectorSubcoreMesh(
        core_axis_name="c", subcore_axis_name="s",
        num_cores=_NC, num_subcores=_NS)
    cedges = chunk_rows * _IDXW
    fn = pl.kernel(
        functools.partial(_gather_add_body, edges_per_w, chunk_rows),
        out_type=jax.ShapeDtypeStruct((ep, _LANES), jnp.float32),
        mesh=mesh,
        compiler_params=pltpu.CompilerParams(use_tc_tiling_on_sc=False),
        scratch_types=[
            pltpu.VMEM((edges_per_w,), jnp.int32),
            pltpu.VMEM((edges_per_w,), jnp.int32),
            pltpu.VMEM((cedges, _LANES), jnp.float32),
            pltpu.VMEM((cedges, _LANES), jnp.float32),
            pltpu.VMEM((cedges, _LANES), jnp.float32),
            pltpu.VMEM((cedges, _LANES), jnp.float32),
            pltpu.SemaphoreType.DMA,
            pltpu.SemaphoreType.DMA,
            pltpu.SemaphoreType.DMA,
            pltpu.SemaphoreType.DMA,
        ],
    )
    return fn(table, si, di)


_PK = 128 // _LANES
_PBLK = _BLK // _PK


def _attn_body(l_ref, anc_ref, out_ref):
    lp = l_ref[...]
    lp = jnp.maximum(lp, 0.01 * lp)
    lane = lax.broadcasted_iota(jnp.int32, lp.shape, 1) % _LANES
    m = lp
    for k in (1, 2, 4, 8):
        r = jnp.where(lane < _LANES - k,
                      pltpu.roll(m, 128 - k, axis=1),
                      pltpu.roll(m, _LANES - k, axis=1))
        m = jnp.maximum(m, r)
    e = jnp.exp(lp - m)
    s = e
    for k in (1, 2, 4, 8):
        r = jnp.where(lane < _LANES - k,
                      pltpu.roll(s, 128 - k, axis=1),
                      pltpu.roll(s, _LANES - k, axis=1))
        s = s + r
    att = e / s
    anc = anc_ref[...]
    for k in range(_PK):
        out_ref[:, k, :] = jnp.dot(
            att[:, k * _LANES:(k + 1) * _LANES], anc,
            preferred_element_type=jnp.float32,
            precision=jax.lax.Precision.HIGHEST)


def _attn(logits, anchor_pad, etot, ep, d):
    nblk = ep // _BLK
    prows = etot // _PK
    lp = logits.reshape(ep // _PK, 128)
    out3 = pl.pallas_call(
        _attn_body,
        grid=(nblk,),
        in_specs=[
            pl.BlockSpec((_PBLK, 128), lambda i: (i, 0)),
            pl.BlockSpec((_LANES, d), lambda i: (0, 0)),
        ],
        out_specs=pl.BlockSpec((_PBLK, _PK, d), lambda i: (i, 0, 0)),
        out_shape=jax.ShapeDtypeStruct((prows, _PK, d), jnp.float32),
    )(lp, anchor_pad)
    return out3.reshape(etot, d)


def kernel(x, edge_index, layer, W, b_lin, anchor):
    del layer
    n, d = x.shape
    a = anchor.shape[0]
    e = edge_index.shape[1]
    etot = e + n

    quantum = _NW * _IDXW
    ep = -(-etot // quantum) * quantum

    table = _make_tables(x, W, b_lin)

    sl = jnp.arange(n, dtype=jnp.int32)
    si = jnp.concatenate([edge_index[0].astype(jnp.int32), sl])
    di = jnp.concatenate([edge_index[1].astype(jnp.int32), sl]) + n
    si = jnp.pad(si, (0, ep - etot))
    di = jnp.pad(di, (0, ep - etot), constant_values=n)

    rows_per_w = (ep // _IDXW) // _NW
    chunk_rows = 1
    for c in (9, 8, 12, 6, 4, 3, 2):
        if rows_per_w % c == 0:
            chunk_rows = c
            break
    logits = _gather_add(table, si, di, ep, chunk_rows)

    anchor_pad = jnp.pad(anchor.astype(jnp.float32), ((0, _LANES - a), (0, 0)))
    return _attn(logits, anchor_pad, etot, ep, d)


# speedup vs baseline: 1.2628x; 1.2628x over previous
import functools

import jax
import jax.numpy as jnp
from jax import lax
from jax.experimental import pallas as pl
from jax.experimental.pallas import tpu as pltpu
from jax.experimental.pallas import tpu_sc as plsc

_LANES = 16
_NC = 2
_NS = 16
_NW = _NC * _NS
_IDXW = 128
_BLK = 4096


def _proj_body(n, x_ref, w_ref, bp_ref, out_ref):
    x = x_ref[...]
    p = jnp.dot(x, w_ref[...], preferred_element_type=jnp.float32)
    out_ref[0:n, :] = p[:, 0:_LANES] + bp_ref[...]
    out_ref[n : 2 * n, :] = p[:, _LANES : 2 * _LANES]


def _make_tables(x, W, b_lin):
    n, d = x.shape
    a = W.shape[1]
    ws = jnp.pad(W[:d], ((0, 0), (0, _LANES - a)))
    wd = jnp.pad(W[d:], ((0, 0), (0, _LANES - a)))
    wfull = jnp.concatenate([ws, wd], axis=1)
    bp = jnp.full((1, _LANES), -1e30, dtype=jnp.float32)
    bp = bp.at[0, :a].set(b_lin.astype(jnp.float32))
    return pl.pallas_call(
        functools.partial(_proj_body, n),
        out_shape=jax.ShapeDtypeStruct((2 * n, _LANES), jnp.float32),
    )(x.astype(jnp.float32), wfull, bp)


def _gather_add_body(edges_per_w, chunk_rows, table_ref, si_ref, di_ref,
                     out_ref, si_v, di_v, a0_v, b0_v, a1_v, b1_v,
                     g0_sem, g1_sem, s0_sem, s1_sem):
    wid = lax.axis_index("s") * _NC + lax.axis_index("c")
    e0 = wid * edges_per_w
    pltpu.sync_copy(si_ref.at[pl.ds(e0, edges_per_w)], si_v)
    pltpu.sync_copy(di_ref.at[pl.ds(e0, edges_per_w)], di_v)
    cedges = chunk_rows * _IDXW
    nchunks = edges_per_w // cedges
    a_bufs, b_bufs = (a0_v, a1_v), (b0_v, b1_v)
    g_sems, s_sems = (g0_sem, g1_sem), (s0_sem, s1_sem)

    def fire(ch, slot):
        ds = []
        for j in range(chunk_rows):
            o = ch * cedges + j * _IDXW
            ds.append(pltpu.async_copy(
                table_ref.at[si_v.at[pl.ds(o, _IDXW)]],
                a_bufs[slot].at[pl.ds(j * _IDXW, _IDXW)], g_sems[slot]))
            ds.append(pltpu.async_copy(
                table_ref.at[di_v.at[pl.ds(o, _IDXW)]],
                b_bufs[slot].at[pl.ds(j * _IDXW, _IDXW)], g_sems[slot]))
        return ds

    pend_g = {0: fire(0, 0)}
    pend_s = {}
    for ch in range(nchunks):
        slot = ch % 2
        nxt = 1 - slot
        if ch + 1 < nchunks:
            if nxt in pend_s:
                pend_s.pop(nxt).wait()
            pend_g[nxt] = fire(ch + 1, nxt)
        for dsc in pend_g.pop(slot):
            dsc.wait()
        a_v, b_v = a_bufs[slot], b_bufs[slot]

        def _add(i, _):
            a_v[i, :] = a_v[i, :] + b_v[i, :]
            return 0

        lax.fori_loop(0, cedges, _add, 0, unroll=8)
        pend_s[slot] = pltpu.async_copy(
            a_v, out_ref.at[pl.ds(e0 + ch * cedges, cedges)], s_sems[slot])
    for dsc in pend_s.values():
        dsc.wait()


def _gather_add(table, si, di, ep, chunk_rows):
    edges_per_w = ep // _NW
    mesh = plsc.VectorSubcoreMesh(
        core_axis_name="c", subcore_axis_name="s",
        num_cores=_NC, num_subcores=_NS)
    cedges = chunk_rows * _IDXW
    fn = pl.kernel(
        functools.partial(_gather_add_body, edges_per_w, chunk_rows),
        out_type=jax.ShapeDtypeStruct((ep, _LANES), jnp.float32),
        mesh=mesh,
        compiler_params=pltpu.CompilerParams(use_tc_tiling_on_sc=False),
        scratch_types=[
            pltpu.VMEM((edges_per_w,), jnp.int32),
            pltpu.VMEM((edges_per_w,), jnp.int32),
            pltpu.VMEM((cedges, _LANES), jnp.float32),
            pltpu.VMEM((cedges, _LANES), jnp.float32),
            pltpu.VMEM((cedges, _LANES), jnp.float32),
            pltpu.VMEM((cedges, _LANES), jnp.float32),
            pltpu.SemaphoreType.DMA,
            pltpu.SemaphoreType.DMA,
            pltpu.SemaphoreType.DMA,
            pltpu.SemaphoreType.DMA,
        ],
    )
    return fn(table, si, di)


_PK = 128 // _LANES
_PBLK = _BLK // _PK


def _attn_body(l_ref, anc_ref, out_ref):
    lp = l_ref[...]
    lp = jnp.maximum(lp, 0.01 * lp)
    lane = lax.broadcasted_iota(jnp.int32, lp.shape, 1) % _LANES
    m = lp
    for k in (1, 2, 4, 8):
        r = jnp.where(lane < _LANES - k,
                      pltpu.roll(m, 128 - k, axis=1),
                      pltpu.roll(m, _LANES - k, axis=1))
        m = jnp.maximum(m, r)
    e = jnp.exp(lp - m)
    s = e
    for k in (1, 2, 4, 8):
        r = jnp.where(lane < _LANES - k,
                      pltpu.roll(s, 128 - k, axis=1),
                      pltpu.roll(s, _LANES - k, axis=1))
        s = s + r
    att = e / s
    anc = anc_ref[...]
    for k in range(_PK):
        out_ref[:, k, :] = jnp.dot(
            att[:, k * _LANES:(k + 1) * _LANES], anc,
            preferred_element_type=jnp.float32)


def _attn(logits, anchor_pad, etot, ep, d):
    nblk = ep // _BLK
    prows = etot // _PK
    lp = logits.reshape(ep // _PK, 128)
    out3 = pl.pallas_call(
        _attn_body,
        grid=(nblk,),
        in_specs=[
            pl.BlockSpec((_PBLK, 128), lambda i: (i, 0)),
            pl.BlockSpec((_LANES, d), lambda i: (0, 0)),
        ],
        out_specs=pl.BlockSpec((_PBLK, _PK, d), lambda i: (i, 0, 0)),
        out_shape=jax.ShapeDtypeStruct((prows, _PK, d), jnp.float32),
    )(lp, anchor_pad)
    return out3.reshape(etot, d)


def kernel(x, edge_index, layer, W, b_lin, anchor):
    del layer
    n, d = x.shape
    a = anchor.shape[0]
    e = edge_index.shape[1]
    etot = e + n

    quantum = _NW * _IDXW
    ep = -(-etot // quantum) * quantum

    table = _make_tables(x, W, b_lin)

    sl = jnp.arange(n, dtype=jnp.int32)
    si = jnp.concatenate([edge_index[0].astype(jnp.int32), sl])
    di = jnp.concatenate([edge_index[1].astype(jnp.int32), sl]) + n
    si = jnp.pad(si, (0, ep - etot))
    di = jnp.pad(di, (0, ep - etot), constant_values=n)

    rows_per_w = (ep // _IDXW) // _NW
    chunk_rows = 1
    for c in (9, 8, 12, 6, 4, 3, 2):
        if rows_per_w % c == 0:
            chunk_rows = c
            break
    logits = _gather_add(table, si, di, ep, chunk_rows)

    anchor_pad = jnp.pad(anchor.astype(jnp.float32), ((0, _LANES - a), (0, 0)))
    return _attn(logits, anchor_pad, etot, ep, d)


# speedup vs baseline: 1.2832x; 1.0161x over previous
import functools

import jax
import jax.numpy as jnp
from jax import lax
from jax.experimental import pallas as pl
from jax.experimental.pallas import tpu as pltpu
from jax.experimental.pallas import tpu_sc as plsc

_LANES = 16
_NC = 2
_NS = 16
_NW = _NC * _NS
_IDXW = 128
_BLK = 4096


def _proj_body(n, x_ref, w_ref, bp_ref, out_ref):
    x = x_ref[...]
    p = jnp.dot(x, w_ref[...], preferred_element_type=jnp.float32)
    out_ref[0:n, :] = p[:, 0:_LANES] + bp_ref[...]
    out_ref[n : 2 * n, :] = p[:, _LANES : 2 * _LANES]


def _make_tables(x, W, b_lin):
    n, d = x.shape
    a = W.shape[1]
    ws = jnp.pad(W[:d], ((0, 0), (0, _LANES - a)))
    wd = jnp.pad(W[d:], ((0, 0), (0, _LANES - a)))
    wfull = jnp.concatenate([ws, wd], axis=1)
    bp = jnp.full((1, _LANES), -1e30, dtype=jnp.float32)
    bp = bp.at[0, :a].set(b_lin.astype(jnp.float32))
    return pl.pallas_call(
        functools.partial(_proj_body, n),
        out_shape=jax.ShapeDtypeStruct((2 * n, _LANES), jnp.float32),
    )(x.astype(jnp.float32), wfull, bp)


def _gather_add_body(edges_per_w, chunk_rows, table_ref, si_ref, di_ref,
                     out_ref, si_v, di_v, a0_v, b0_v, a1_v, b1_v,
                     g0_sem, g1_sem, s0_sem, s1_sem):
    wid = lax.axis_index("s") * _NC + lax.axis_index("c")
    e0 = wid * edges_per_w
    pltpu.sync_copy(si_ref.at[pl.ds(e0, edges_per_w)], si_v)
    pltpu.sync_copy(di_ref.at[pl.ds(e0, edges_per_w)], di_v)
    cedges = chunk_rows * _IDXW
    nchunks = edges_per_w // cedges
    a_bufs, b_bufs = (a0_v, a1_v), (b0_v, b1_v)
    g_sems, s_sems = (g0_sem, g1_sem), (s0_sem, s1_sem)

    def fire(ch, slot):
        ds = []
        for j in range(chunk_rows):
            o = ch * cedges + j * _IDXW
            ds.append(pltpu.async_copy(
                table_ref.at[si_v.at[pl.ds(o, _IDXW)]],
                a_bufs[slot].at[pl.ds(j * _IDXW, _IDXW)], g_sems[slot]))
            ds.append(pltpu.async_copy(
                table_ref.at[di_v.at[pl.ds(o, _IDXW)]],
                b_bufs[slot].at[pl.ds(j * _IDXW, _IDXW)], g_sems[slot]))
        return ds

    pend_g = {0: fire(0, 0)}
    pend_s = {}
    for ch in range(nchunks):
        slot = ch % 2
        nxt = 1 - slot
        if ch + 1 < nchunks:
            if nxt in pend_s:
                pend_s.pop(nxt).wait()
            pend_g[nxt] = fire(ch + 1, nxt)
        for dsc in pend_g.pop(slot):
            dsc.wait()
        a_v, b_v = a_bufs[slot], b_bufs[slot]

        def _add(i, _):
            a_v[i, :] = a_v[i, :] + b_v[i, :]
            return 0

        lax.fori_loop(0, cedges, _add, 0, unroll=8)
        pend_s[slot] = pltpu.async_copy(
            a_v, out_ref.at[pl.ds(e0 + ch * cedges, cedges)], s_sems[slot])
    for dsc in pend_s.values():
        dsc.wait()


def _gather_add(table, si, di, ep, chunk_rows):
    edges_per_w = ep // _NW
    mesh = plsc.VectorSubcoreMesh(
        core_axis_name="c", subcore_axis_name="s",
        num_cores=_NC, num_subcores=_NS)
    cedges = chunk_rows * _IDXW
    fn = pl.kernel(
        functools.partial(_gather_add_body, edges_per_w, chunk_rows),
        out_type=jax.ShapeDtypeStruct((ep, _LANES), jnp.float32),
        mesh=mesh,
        compiler_params=pltpu.CompilerParams(use_tc_tiling_on_sc=False),
        scratch_types=[
            pltpu.VMEM((edges_per_w,), jnp.int32),
            pltpu.VMEM((edges_per_w,), jnp.int32),
            pltpu.VMEM((cedges, _LANES), jnp.float32),
            pltpu.VMEM((cedges, _LANES), jnp.float32),
            pltpu.VMEM((cedges, _LANES), jnp.float32),
            pltpu.VMEM((cedges, _LANES), jnp.float32),
            pltpu.SemaphoreType.DMA,
            pltpu.SemaphoreType.DMA,
            pltpu.SemaphoreType.DMA,
            pltpu.SemaphoreType.DMA,
        ],
    )
    return fn(table, si, di)


_PK = 128 // _LANES
_PBLK = _BLK // _PK


def _attn_body(l_ref, anc_ref, out_ref):
    lp = l_ref[...]
    lp = jnp.maximum(lp, 0.01 * lp)
    lane = lax.broadcasted_iota(jnp.int32, lp.shape, 1) % _LANES
    m = lp
    for k in (1, 2, 4, 8):
        r = jnp.where(lane < _LANES - k,
                      pltpu.roll(m, 128 - k, axis=1),
                      pltpu.roll(m, _LANES - k, axis=1))
        m = jnp.maximum(m, r)
    e = jnp.exp(lp - m)
    s = e
    for k in (1, 2, 4, 8):
        r = jnp.where(lane < _LANES - k,
                      pltpu.roll(s, 128 - k, axis=1),
                      pltpu.roll(s, _LANES - k, axis=1))
        s = s + r
    att = e / s
    anc = anc_ref[...]
    for k in range(_PK):
        out_ref[pl.ds(k * _PBLK, _PBLK), :] = jnp.dot(
            att[:, k * _LANES:(k + 1) * _LANES], anc,
            preferred_element_type=jnp.float32)


def _attn(logits, anchor_pad, etot, ep, d):
    nblk = ep // _BLK
    lp = logits.reshape(ep // _PK, 128)
    return pl.pallas_call(
        _attn_body,
        grid=(nblk,),
        in_specs=[
            pl.BlockSpec((_PBLK, 128), lambda i: (i, 0)),
            pl.BlockSpec((_LANES, d), lambda i: (0, 0)),
        ],
        out_specs=pl.BlockSpec((_BLK, d), lambda i: (i, 0)),
        out_shape=jax.ShapeDtypeStruct((etot, d), jnp.float32),
    )(lp, anchor_pad)


def kernel(x, edge_index, layer, W, b_lin, anchor):
    del layer
    n, d = x.shape
    a = anchor.shape[0]
    e = edge_index.shape[1]
    etot = e + n

    quantum = _NW * _IDXW
    ep = -(-etot // quantum) * quantum

    table = _make_tables(x, W, b_lin)

    sl = jnp.arange(n, dtype=jnp.int32)
    si = jnp.concatenate([edge_index[0].astype(jnp.int32), sl])
    di = jnp.concatenate([edge_index[1].astype(jnp.int32), sl]) + n
    si = jnp.pad(si, (0, ep - etot))
    di = jnp.pad(di, (0, ep - etot), constant_values=n)
    nblk = ep // _BLK
    si = si.reshape(nblk, _PK, _PBLK).swapaxes(1, 2).reshape(ep)
    di = di.reshape(nblk, _PK, _PBLK).swapaxes(1, 2).reshape(ep)

    rows_per_w = (ep // _IDXW) // _NW
    chunk_rows = 1
    for c in (9, 8, 12, 6, 4, 3, 2):
        if rows_per_w % c == 0:
            chunk_rows = c
            break
    logits = _gather_add(table, si, di, ep, chunk_rows)

    anchor_pad = jnp.pad(anchor.astype(jnp.float32), ((0, _LANES - a), (0, 0)))
    return _attn(logits, anchor_pad, etot, ep, d)


# speedup vs baseline: 1.3231x; 1.0311x over previous
import functools

import jax
import jax.numpy as jnp
from jax import lax
from jax.experimental import pallas as pl
from jax.experimental.pallas import tpu as pltpu
from jax.experimental.pallas import tpu_sc as plsc

_LANES = 16
_NC = 2
_NS = 16
_NW = _NC * _NS
_IDXW = 128
_BLK = 10368


def _proj_body(n, x_ref, w_ref, bp_ref, out_ref):
    x = x_ref[...]
    p = jnp.dot(x, w_ref[...], preferred_element_type=jnp.float32)
    out_ref[0:n, :] = p[:, 0:_LANES] + bp_ref[...]
    out_ref[n : 2 * n, :] = p[:, _LANES : 2 * _LANES]


def _make_tables(x, W, b_lin):
    n, d = x.shape
    a = W.shape[1]
    ws = jnp.pad(W[:d], ((0, 0), (0, _LANES - a)))
    wd = jnp.pad(W[d:], ((0, 0), (0, _LANES - a)))
    wfull = jnp.concatenate([ws, wd], axis=1)
    bp = jnp.full((1, _LANES), -1e30, dtype=jnp.float32)
    bp = bp.at[0, :a].set(b_lin.astype(jnp.float32))
    return pl.pallas_call(
        functools.partial(_proj_body, n),
        out_shape=jax.ShapeDtypeStruct((2 * n, _LANES), jnp.float32),
    )(x.astype(jnp.float32), wfull, bp)


def _gather_add_body(edges_per_w, chunk_rows, table_ref, si_ref, di_ref,
                     out_ref, si_v, di_v, a0_v, b0_v, a1_v, b1_v,
                     g0_sem, g1_sem, s0_sem, s1_sem):
    wid = lax.axis_index("s") * _NC + lax.axis_index("c")
    e0 = wid * edges_per_w
    pltpu.sync_copy(si_ref.at[pl.ds(e0, edges_per_w)], si_v)
    pltpu.sync_copy(di_ref.at[pl.ds(e0, edges_per_w)], di_v)
    cedges = chunk_rows * _IDXW
    nchunks = edges_per_w // cedges
    a_bufs, b_bufs = (a0_v, a1_v), (b0_v, b1_v)
    g_sems, s_sems = (g0_sem, g1_sem), (s0_sem, s1_sem)

    def fire(ch, slot):
        ds = []
        for j in range(chunk_rows):
            o = ch * cedges + j * _IDXW
            ds.append(pltpu.async_copy(
                table_ref.at[si_v.at[pl.ds(o, _IDXW)]],
                a_bufs[slot].at[pl.ds(j * _IDXW, _IDXW)], g_sems[slot]))
            ds.append(pltpu.async_copy(
                table_ref.at[di_v.at[pl.ds(o, _IDXW)]],
                b_bufs[slot].at[pl.ds(j * _IDXW, _IDXW)], g_sems[slot]))
        return ds

    pend_g = {0: fire(0, 0)}
    pend_s = {}
    for ch in range(nchunks):
        slot = ch % 2
        nxt = 1 - slot
        if ch + 1 < nchunks:
            if nxt in pend_s:
                pend_s.pop(nxt).wait()
            pend_g[nxt] = fire(ch + 1, nxt)
        for dsc in pend_g.pop(slot):
            dsc.wait()
        a_v, b_v = a_bufs[slot], b_bufs[slot]

        def _add(i, _):
            a_v[i, :] = a_v[i, :] + b_v[i, :]
            return 0

        lax.fori_loop(0, cedges, _add, 0, unroll=8)
        pend_s[slot] = pltpu.async_copy(
            a_v, out_ref.at[pl.ds(e0 + ch * cedges, cedges)], s_sems[slot])
    for dsc in pend_s.values():
        dsc.wait()


def _gather_add(table, si, di, ep, chunk_rows):
    edges_per_w = ep // _NW
    mesh = plsc.VectorSubcoreMesh(
        core_axis_name="c", subcore_axis_name="s",
        num_cores=_NC, num_subcores=_NS)
    cedges = chunk_rows * _IDXW
    fn = pl.kernel(
        functools.partial(_gather_add_body, edges_per_w, chunk_rows),
        out_type=jax.ShapeDtypeStruct((ep, _LANES), jnp.float32),
        mesh=mesh,
        compiler_params=pltpu.CompilerParams(use_tc_tiling_on_sc=False),
        scratch_types=[
            pltpu.VMEM((edges_per_w,), jnp.int32),
            pltpu.VMEM((edges_per_w,), jnp.int32),
            pltpu.VMEM((cedges, _LANES), jnp.float32),
            pltpu.VMEM((cedges, _LANES), jnp.float32),
            pltpu.VMEM((cedges, _LANES), jnp.float32),
            pltpu.VMEM((cedges, _LANES), jnp.float32),
            pltpu.SemaphoreType.DMA,
            pltpu.SemaphoreType.DMA,
            pltpu.SemaphoreType.DMA,
            pltpu.SemaphoreType.DMA,
        ],
    )
    return fn(table, si, di)


_PK = 128 // _LANES
_PBLK = _BLK // _PK


def _attn_body(l_ref, anc_ref, out_ref):
    lp = l_ref[...]
    lp = jnp.maximum(lp, 0.01 * lp)
    lane = lax.broadcasted_iota(jnp.int32, lp.shape, 1) % _LANES
    m = lp
    for k in (1, 2, 4, 8):
        r = jnp.where(lane < _LANES - k,
                      pltpu.roll(m, 128 - k, axis=1),
                      pltpu.roll(m, _LANES - k, axis=1))
        m = jnp.maximum(m, r)
    e = jnp.exp(lp - m)
    s = e
    for k in (1, 2, 4, 8):
        r = jnp.where(lane < _LANES - k,
                      pltpu.roll(s, 128 - k, axis=1),
                      pltpu.roll(s, _LANES - k, axis=1))
        s = s + r
    att = e / s
    anc = anc_ref[...]
    for k in range(_PK):
        out_ref[pl.ds(k * _PBLK, _PBLK), :] = jnp.dot(
            att[:, k * _LANES:(k + 1) * _LANES], anc,
            preferred_element_type=jnp.float32)


def _attn(logits, anchor_pad, etot, ep, d):
    nblk = ep // _BLK
    lp = logits.reshape(ep // _PK, 128)
    return pl.pallas_call(
        _attn_body,
        grid=(nblk,),
        in_specs=[
            pl.BlockSpec((_PBLK, 128), lambda i: (i, 0)),
            pl.BlockSpec((_LANES, d), lambda i: (0, 0)),
        ],
        out_specs=pl.BlockSpec((_BLK, d), lambda i: (i, 0)),
        out_shape=jax.ShapeDtypeStruct((etot, d), jnp.float32),
    )(lp, anchor_pad)


def kernel(x, edge_index, layer, W, b_lin, anchor):
    del layer
    n, d = x.shape
    a = anchor.shape[0]
    e = edge_index.shape[1]
    etot = e + n

    import math
    quantum = math.lcm(_NW * _IDXW, _BLK)
    ep = -(-etot // quantum) * quantum

    table = _make_tables(x, W, b_lin)

    sl = jnp.arange(n, dtype=jnp.int32)
    si = jnp.concatenate([edge_index[0].astype(jnp.int32), sl])
    di = jnp.concatenate([edge_index[1].astype(jnp.int32), sl]) + n
    si = jnp.pad(si, (0, ep - etot))
    di = jnp.pad(di, (0, ep - etot), constant_values=n)
    nblk = ep // _BLK
    si = si.reshape(nblk, _PK, _PBLK).swapaxes(1, 2).reshape(ep)
    di = di.reshape(nblk, _PK, _PBLK).swapaxes(1, 2).reshape(ep)

    rows_per_w = (ep // _IDXW) // _NW
    chunk_rows = 1
    for c in (9, 8, 12, 6, 4, 3, 2):
        if rows_per_w % c == 0:
            chunk_rows = c
            break
    logits = _gather_add(table, si, di, ep, chunk_rows)

    anchor_pad = jnp.pad(anchor.astype(jnp.float32), ((0, _LANES - a), (0, 0)))
    return _attn(logits, anchor_pad, etot, ep, d)


# speedup vs baseline: 1.5417x; 1.1653x over previous
import functools

import jax
import jax.numpy as jnp
from jax import lax
from jax.experimental import pallas as pl
from jax.experimental.pallas import tpu as pltpu
from jax.experimental.pallas import tpu_sc as plsc

_LANES = 16
_NC = 2
_NS = 16
_NW = _NC * _NS
_IDXW = 128
_BLK = 10368


def _proj_body(n, x_ref, w_ref, bp_ref, out_ref):
    x = x_ref[...]
    p = jnp.dot(x, w_ref[...], preferred_element_type=jnp.float32)
    out_ref[0:n, :] = p[:, 0:_LANES] + bp_ref[...]
    out_ref[n : 2 * n, :] = p[:, _LANES : 2 * _LANES]


def _make_tables(x, W, b_lin):
    n, d = x.shape
    a = W.shape[1]
    ws = jnp.pad(W[:d], ((0, 0), (0, _LANES - a)))
    wd = jnp.pad(W[d:], ((0, 0), (0, _LANES - a)))
    wfull = jnp.concatenate([ws, wd], axis=1)
    bp = jnp.full((1, _LANES), -1e30, dtype=jnp.float32)
    bp = bp.at[0, :a].set(b_lin.astype(jnp.float32))
    return pl.pallas_call(
        functools.partial(_proj_body, n),
        out_shape=jax.ShapeDtypeStruct((2 * n, _LANES), jnp.float32),
    )(x.astype(jnp.float32), wfull, bp)


_NSLOT = 3


def _gather_add_body(edges_per_w, chunk_rows, table_ref, si_ref, di_ref,
                     out_ref, si_v, di_v, a_bufs, b_bufs, sp_bufs, dp_bufs,
                     o_bufs, g_sems, s_sems):
    wid = lax.axis_index("s") * _NC + lax.axis_index("c")
    e0 = wid * edges_per_w
    pltpu.sync_copy(si_ref.at[pl.ds(e0, edges_per_w)], si_v)
    pltpu.sync_copy(di_ref.at[pl.ds(e0, edges_per_w)], di_v)
    cedges = chunk_rows * _IDXW
    nchunks = edges_per_w // cedges
    prows_w = edges_per_w // _PK
    crows = cedges // _PK
    nsub = crows // _LANES

    def fire(ch, slot):
        sp, dp = sp_bufs[slot], dp_bufs[slot]

        def _stage(t, _):
            m = t // nsub
            j = t - m * nsub
            src = prows_w * m + crows * ch + _LANES * j
            sp[pl.ds(_LANES * t, _LANES)] = si_v[pl.ds(src, _LANES)]
            dp[pl.ds(_LANES * t, _LANES)] = di_v[pl.ds(src, _LANES)]
            return 0

        lax.fori_loop(0, _PK * nsub, _stage, 0, unroll=4)
        ds = []
        for j in range(chunk_rows):
            o = j * _IDXW
            ds.append(pltpu.async_copy(
                table_ref.at[sp.at[pl.ds(o, _IDXW)]],
                a_bufs[slot].at[pl.ds(j * _IDXW, _IDXW)], g_sems[slot]))
            ds.append(pltpu.async_copy(
                table_ref.at[dp.at[pl.ds(o, _IDXW)]],
                b_bufs[slot].at[pl.ds(j * _IDXW, _IDXW)], g_sems[slot]))
        return ds

    pend_g, pend_s = {}, {}
    for c in range(min(_NSLOT - 1, nchunks)):
        pend_g[c % _NSLOT] = fire(c, c % _NSLOT)
    for ch in range(nchunks):
        slot = ch % _NSLOT
        nf = ch + _NSLOT - 1
        if nf < nchunks:
            fs = nf % _NSLOT
            if fs in pend_s:
                pend_s.pop(fs).wait()
            pend_g[fs] = fire(nf, fs)
        for dsc in pend_g.pop(slot):
            dsc.wait()
        a_v, b_v, o_v = a_bufs[slot], b_bufs[slot], o_bufs[slot]

        def _addperm(i, _):
            q = crows * (i & (_PK - 1)) + (i >> 3)
            o_v[i, :] = a_v[q, :] + b_v[q, :]
            return 0

        lax.fori_loop(0, cedges, _addperm, 0, unroll=8)
        if slot in pend_s:
            pend_s.pop(slot).wait()
        pend_s[slot] = pltpu.async_copy(
            o_v, out_ref.at[pl.ds(e0 + ch * cedges, cedges)], s_sems[slot])
    for dsc in pend_s.values():
        dsc.wait()


def _ga_entry(edges_per_w, chunk_rows, table_ref, si_ref, di_ref, out_ref,
              si_v, di_v,
              a0, b0, sp0, dp0, o0, a1, b1, sp1, dp1, o1,
              a2, b2, sp2, dp2, o2,
              g0, g1, g2, s0, s1, s2):
    _gather_add_body(edges_per_w, chunk_rows, table_ref, si_ref, di_ref,
                     out_ref, si_v, di_v,
                     (a0, a1, a2), (b0, b1, b2), (sp0, sp1, sp2),
                     (dp0, dp1, dp2), (o0, o1, o2),
                     (g0, g1, g2), (s0, s1, s2))


def _gather_add(table, si, di, ep, chunk_rows):
    edges_per_w = ep // _NW
    mesh = plsc.VectorSubcoreMesh(
        core_axis_name="c", subcore_axis_name="s",
        num_cores=_NC, num_subcores=_NS)
    cedges = chunk_rows * _IDXW
    slot_scratch = [
        pltpu.VMEM((cedges, _LANES), jnp.float32),
        pltpu.VMEM((cedges, _LANES), jnp.float32),
        pltpu.VMEM((cedges,), jnp.int32),
        pltpu.VMEM((cedges,), jnp.int32),
        pltpu.VMEM((cedges, _LANES), jnp.float32),
    ]
    fn = pl.kernel(
        functools.partial(_ga_entry, edges_per_w, chunk_rows),
        out_type=jax.ShapeDtypeStruct((ep, _LANES), jnp.float32),
        mesh=mesh,
        compiler_params=pltpu.CompilerParams(use_tc_tiling_on_sc=False),
        scratch_types=(
            [pltpu.VMEM((edges_per_w,), jnp.int32),
             pltpu.VMEM((edges_per_w,), jnp.int32)]
            + slot_scratch * _NSLOT
            + [pltpu.SemaphoreType.DMA] * (2 * _NSLOT)
        ),
    )
    return fn(table, si, di)


_PK = 128 // _LANES
_PBLK = _BLK // _PK


def _attn_body(l_ref, anc_ref, out_ref):
    lp = l_ref[...]
    lp = jnp.maximum(lp, 0.01 * lp)
    lane = lax.broadcasted_iota(jnp.int32, lp.shape, 1) % _LANES
    m = lp
    for k in (1, 2, 4, 8):
        r = jnp.where(lane < _LANES - k,
                      pltpu.roll(m, 128 - k, axis=1),
                      pltpu.roll(m, _LANES - k, axis=1))
        m = jnp.maximum(m, r)
    e = jnp.exp(lp - m)
    s = e
    for k in (1, 2, 4, 8):
        r = jnp.where(lane < _LANES - k,
                      pltpu.roll(s, 128 - k, axis=1),
                      pltpu.roll(s, _LANES - k, axis=1))
        s = s + r
    att = e / s
    anc = anc_ref[...]
    for k in range(_PK):
        out_ref[pl.ds(k * _PBLK, _PBLK), :] = jnp.dot(
            att[:, k * _LANES:(k + 1) * _LANES], anc,
            preferred_element_type=jnp.float32)


def _attn(logits, anchor_pad, etot, ep, d):
    nblk = ep // _BLK
    lp = logits.reshape(ep // _PK, 128)
    return pl.pallas_call(
        _attn_body,
        grid=(nblk,),
        in_specs=[
            pl.BlockSpec((_PBLK, 128), lambda i: (i, 0)),
            pl.BlockSpec((_LANES, d), lambda i: (0, 0)),
        ],
        out_specs=pl.BlockSpec((_BLK, d), lambda i: (i, 0)),
        out_shape=jax.ShapeDtypeStruct((etot, d), jnp.float32),
    )(lp, anchor_pad)


def kernel(x, edge_index, layer, W, b_lin, anchor):
    del layer
    n, d = x.shape
    a = anchor.shape[0]
    e = edge_index.shape[1]
    etot = e + n

    import math
    quantum = math.lcm(_NW * _IDXW, _BLK)
    ep = -(-etot // quantum) * quantum

    table = _make_tables(x, W, b_lin)

    sl = jnp.arange(n, dtype=jnp.int32)
    si = jnp.concatenate([edge_index[0].astype(jnp.int32), sl])
    di = jnp.concatenate([edge_index[1].astype(jnp.int32), sl]) + n
    si = jnp.pad(si, (0, ep - etot))
    di = jnp.pad(di, (0, ep - etot), constant_values=n)

    rows_per_w = (ep // _IDXW) // _NW
    chunk_rows = 1
    for c in (3, 4, 2):
        if rows_per_w % c == 0:
            chunk_rows = c
            break
    logits = _gather_add(table, si, di, ep, chunk_rows)

    anchor_pad = jnp.pad(anchor.astype(jnp.float32), ((0, _LANES - a), (0, 0)))
    return _attn(logits, anchor_pad, etot, ep, d)


# speedup vs baseline: 1.5826x; 1.0265x over previous
import functools

import jax
import jax.numpy as jnp
from jax import lax
from jax.experimental import pallas as pl
from jax.experimental.pallas import tpu as pltpu
from jax.experimental.pallas import tpu_sc as plsc

_LANES = 16
_NC = 2
_NS = 16
_NW = _NC * _NS
_IDXW = 128
_BLK = 10368


def _proj_body(n, x_ref, w_ref, bp_ref, out_ref):
    x = x_ref[...]
    p = jnp.dot(x, w_ref[...], preferred_element_type=jnp.float32)
    out_ref[0:n, :] = p[:, 0:_LANES] + bp_ref[...]
    out_ref[n : 2 * n, :] = p[:, _LANES : 2 * _LANES]


def _make_tables(x, W, b_lin):
    n, d = x.shape
    a = W.shape[1]
    ws = jnp.pad(W[:d], ((0, 0), (0, _LANES - a)))
    wd = jnp.pad(W[d:], ((0, 0), (0, _LANES - a)))
    wfull = jnp.concatenate([ws, wd], axis=1)
    bp = jnp.full((1, _LANES), -1e30, dtype=jnp.float32)
    bp = bp.at[0, :a].set(b_lin.astype(jnp.float32))
    return pl.pallas_call(
        functools.partial(_proj_body, n),
        out_shape=jax.ShapeDtypeStruct((2 * n, _LANES), jnp.float32),
    )(x.astype(jnp.float32), wfull, bp)


_NSLOT = 3


def _gather_add_body(edges_per_w, chunk_rows, n_nodes, e_real, etot,
                     table_ref, ei_ref, out_ref, si_v, di_v,
                     a_bufs, b_bufs, sp_bufs, dp_bufs,
                     o_bufs, g_sems, s_sems):
    wid = lax.axis_index("s") * _NC + lax.axis_index("c")
    e0 = wid * edges_per_w
    ce0 = jnp.minimum(e0, e_real - edges_per_w)
    delta = e0 - ce0
    pltpu.sync_copy(ei_ref.at[0, pl.ds(ce0, edges_per_w)], si_v)
    pltpu.sync_copy(ei_ref.at[1, pl.ds(ce0, edges_per_w)], di_v)
    cedges = chunk_rows * _IDXW
    nchunks = edges_per_w // cedges
    prows_w = edges_per_w // _PK
    crows = cedges // _PK
    nsub = crows // _LANES
    li16 = lax.broadcasted_iota(jnp.int32, (_LANES,), 0)

    def fire(ch, slot):
        sp, dp = sp_bufs[slot], dp_bufs[slot]

        def _stage(t, _):
            m = t // nsub
            j = t - m * nsub
            src = prows_w * m + crows * ch + _LANES * j
            lsrc = jnp.minimum(src + delta, edges_per_w - _LANES)
            raw_s = si_v[pl.ds(lsrc, _LANES)]
            raw_d = di_v[pl.ds(lsrc, _LANES)]
            gpos = (e0 + src) + li16
            isreal = gpos < e_real
            ispad = gpos >= etot
            selfv = gpos - e_real
            alt = jnp.where(ispad, 0, selfv)
            sp[pl.ds(_LANES * t, _LANES)] = jnp.where(isreal, raw_s, alt)
            dp[pl.ds(_LANES * t, _LANES)] = (
                jnp.where(isreal, raw_d, alt) + n_nodes)
            return 0

        lax.fori_loop(0, _PK * nsub, _stage, 0, unroll=4)
        ds = []
        for j in range(chunk_rows):
            o = j * _IDXW
            ds.append(pltpu.async_copy(
                table_ref.at[sp.at[pl.ds(o, _IDXW)]],
                a_bufs[slot].at[pl.ds(j * _IDXW, _IDXW)], g_sems[slot]))
            ds.append(pltpu.async_copy(
                table_ref.at[dp.at[pl.ds(o, _IDXW)]],
                b_bufs[slot].at[pl.ds(j * _IDXW, _IDXW)], g_sems[slot]))
        return ds

    pend_g, pend_s = {}, {}
    for c in range(min(_NSLOT - 1, nchunks)):
        pend_g[c % _NSLOT] = fire(c, c % _NSLOT)
    for ch in range(nchunks):
        slot = ch % _NSLOT
        nf = ch + _NSLOT - 1
        if nf < nchunks:
            fs = nf % _NSLOT
            if fs in pend_s:
                pend_s.pop(fs).wait()
            pend_g[fs] = fire(nf, fs)
        for dsc in pend_g.pop(slot):
            dsc.wait()
        a_v, b_v, o_v = a_bufs[slot], b_bufs[slot], o_bufs[slot]

        def _addperm(i, _):
            mm = i & (_PK - 1)
            rr = i >> 3
            q = crows * mm + rr
            o_v[rr, pl.ds(_LANES * mm, _LANES)] = a_v[q, :] + b_v[q, :]
            return 0

        lax.fori_loop(0, cedges, _addperm, 0, unroll=8)
        if slot in pend_s:
            pend_s.pop(slot).wait()
        pend_s[slot] = pltpu.async_copy(
            o_v, out_ref.at[pl.ds(wid * prows_w + ch * crows, crows)],
            s_sems[slot])
    for dsc in pend_s.values():
        dsc.wait()


def _ga_entry(edges_per_w, chunk_rows, n_nodes, e_real, etot,
              table_ref, ei_ref, out_ref,
              si_v, di_v,
              a0, b0, sp0, dp0, o0, a1, b1, sp1, dp1, o1,
              a2, b2, sp2, dp2, o2,
              g0, g1, g2, s0, s1, s2):
    _gather_add_body(edges_per_w, chunk_rows, n_nodes, e_real, etot,
                     table_ref, ei_ref, out_ref, si_v, di_v,
                     (a0, a1, a2), (b0, b1, b2), (sp0, sp1, sp2),
                     (dp0, dp1, dp2), (o0, o1, o2),
                     (g0, g1, g2), (s0, s1, s2))


def _gather_add(table, edge_index, ep, chunk_rows, n_nodes, e_real, etot):
    edges_per_w = ep // _NW
    mesh = plsc.VectorSubcoreMesh(
        core_axis_name="c", subcore_axis_name="s",
        num_cores=_NC, num_subcores=_NS)
    cedges = chunk_rows * _IDXW
    slot_scratch = [
        pltpu.VMEM((cedges, _LANES), jnp.float32),
        pltpu.VMEM((cedges, _LANES), jnp.float32),
        pltpu.VMEM((cedges,), jnp.int32),
        pltpu.VMEM((cedges,), jnp.int32),
        pltpu.VMEM((cedges // _PK, 128), jnp.float32),
    ]
    fn = pl.kernel(
        functools.partial(_ga_entry, edges_per_w, chunk_rows,
                          n_nodes, e_real, etot),
        out_type=jax.ShapeDtypeStruct((ep // _PK, 128), jnp.float32),
        mesh=mesh,
        compiler_params=pltpu.CompilerParams(use_tc_tiling_on_sc=False),
        scratch_types=(
            [pltpu.VMEM((edges_per_w,), jnp.int32),
             pltpu.VMEM((edges_per_w,), jnp.int32)]
            + slot_scratch * _NSLOT
            + [pltpu.SemaphoreType.DMA] * (2 * _NSLOT)
        ),
    )
    return fn(table, edge_index)


_PK = 128 // _LANES
_PBLK = _BLK // _PK


def _attn_body(l_ref, anc_ref, out_ref):
    lp = l_ref[...]
    lp = jnp.maximum(lp, 0.01 * lp)
    lane = lax.broadcasted_iota(jnp.int32, lp.shape, 1) % _LANES
    m = lp
    for k in (1, 2, 4, 8):
        r = jnp.where(lane < _LANES - k,
                      pltpu.roll(m, 128 - k, axis=1),
                      pltpu.roll(m, _LANES - k, axis=1))
        m = jnp.maximum(m, r)
    e = jnp.exp(lp - m)
    s = e
    for k in (1, 2, 4, 8):
        r = jnp.where(lane < _LANES - k,
                      pltpu.roll(s, 128 - k, axis=1),
                      pltpu.roll(s, _LANES - k, axis=1))
        s = s + r
    att = e / s
    anc = anc_ref[...]
    for k in range(_PK):
        out_ref[pl.ds(k * _PBLK, _PBLK), :] = jnp.dot(
            att[:, k * _LANES:(k + 1) * _LANES], anc,
            preferred_element_type=jnp.float32)


def _attn(lp, anchor_pad, etot, ep, d):
    nblk = ep // _BLK
    return pl.pallas_call(
        _attn_body,
        grid=(nblk,),
        in_specs=[
            pl.BlockSpec((_PBLK, 128), lambda i: (i, 0)),
            pl.BlockSpec((_LANES, d), lambda i: (0, 0)),
        ],
        out_specs=pl.BlockSpec((_BLK, d), lambda i: (i, 0)),
        out_shape=jax.ShapeDtypeStruct((etot, d), jnp.float32),
    )(lp, anchor_pad)


def kernel(x, edge_index, layer, W, b_lin, anchor):
    del layer
    n, d = x.shape
    a = anchor.shape[0]
    e = edge_index.shape[1]
    etot = e + n

    import math
    quantum = math.lcm(_NW * _IDXW, _BLK)
    ep = -(-etot // quantum) * quantum

    table = _make_tables(x, W, b_lin)

    rows_per_w = (ep // _IDXW) // _NW
    chunk_rows = 1
    for c in (3, 4, 2):
        if rows_per_w % c == 0:
            chunk_rows = c
            break
    logits = _gather_add(table, edge_index.astype(jnp.int32), ep, chunk_rows,
                         n, e, etot)

    anchor_pad = jnp.pad(anchor.astype(jnp.float32), ((0, _LANES - a), (0, 0)))
    return _attn(logits, anchor_pad, etot, ep, d)
